# Initial kernel scaffold; baseline (speedup 1.0000x reference)
import functools

import jax
import jax.numpy as jnp
from jax import lax
from jax.experimental import pallas as pl
from jax.experimental.pallas import tpu as pltpu


def _body(q_ref, k_ref, v_ref, w_ref, out_ref, cnt_ref, *, n_total):
    mb = pl.program_id(1)
    nm = pl.num_programs(1)
    q2 = q_ref[0]
    k2 = k_ref[0]
    logits = lax.dot_general(q2, k2, (((1,), (1,)), ((), ())),
                             preferred_element_type=jnp.float32)
    col_max = jnp.max(logits, axis=0)
    row_iota = lax.broadcasted_iota(jnp.int32, logits.shape, 0)
    is_max = logits == col_max[None, :]
    idx = jnp.min(jnp.where(is_max, row_iota, n_total), axis=0)
    valid = jnp.isfinite(col_max)
    one_hot = ((row_iota == idx[None, :]) & valid[None, :]).astype(jnp.float32)
    vv = lax.dot_general(v_ref[0], w_ref, (((1,), (1,)), ((), ())),
                         preferred_element_type=jnp.float32)
    contrib = lax.dot_general(one_hot, vv, (((1,), (0,)), ((), ())),
                              preferred_element_type=jnp.float32)
    cnts = jnp.sum(one_hot, axis=1, keepdims=True)

    @pl.when(mb == 0)
    def _():
        out_ref[0] = contrib
        cnt_ref[...] = cnts

    @pl.when(mb > 0)
    def _():
        out_ref[0] += contrib
        cnt_ref[...] += cnts

    @pl.when(mb == nm - 1)
    def _():
        c = cnt_ref[...]
        out_ref[0] = out_ref[0] / (jnp.maximum(c, 1.0) + 1e-6)


def kernel(q, k, v, W_v):
    B, N, D = q.shape
    M = k.shape[1]
    MB = min(512, M)
    body = functools.partial(_body, n_total=N)
    return pl.pallas_call(
        body,
        grid=(B, M // MB),
        in_specs=[
            pl.BlockSpec((1, N, D), lambda b, m: (b, 0, 0)),
            pl.BlockSpec((1, MB, D), lambda b, m: (b, m, 0)),
            pl.BlockSpec((1, MB, D), lambda b, m: (b, m, 0)),
            pl.BlockSpec((D, D), lambda b, m: (0, 0)),
        ],
        out_specs=pl.BlockSpec((1, N, D), lambda b, m: (b, 0, 0)),
        out_shape=jax.ShapeDtypeStruct((B, N, D), jnp.float32),
        scratch_shapes=[pltpu.VMEM((N, 1), jnp.float32)],
    )(q, k, v, W_v)


# speedup vs baseline: 1.5152x; 1.5152x over previous
import functools

import jax
import jax.numpy as jnp
from jax import lax
from jax.experimental import pallas as pl
from jax.experimental.pallas import tpu as pltpu


def _body(q_ref, k_ref, v_ref, w_ref, out_ref, cnt_ref, *, n_total):
    mb = pl.program_id(1)
    nm = pl.num_programs(1)
    q2 = q_ref[0]
    k2 = k_ref[0]
    logits = lax.dot_general(q2, k2, (((1,), (1,)), ((), ())),
                             preferred_element_type=jnp.float32)
    col_max = jnp.max(logits, axis=0)
    row_iota = lax.broadcasted_iota(jnp.int32, logits.shape, 0)
    is_max = logits == col_max[None, :]
    idx = jnp.min(jnp.where(is_max, row_iota, n_total), axis=0)
    valid = jnp.isfinite(col_max)
    one_hot = ((row_iota == idx[None, :]) & valid[None, :]).astype(jnp.float32)
    vv = lax.dot_general(v_ref[0], w_ref[...], (((1,), (1,)), ((), ())),
                         preferred_element_type=jnp.float32)
    contrib = lax.dot_general(one_hot, vv, (((1,), (0,)), ((), ())),
                              preferred_element_type=jnp.float32)
    cnts = jnp.sum(one_hot, axis=1, keepdims=True)

    @pl.when(mb == 0)
    def _():
        out_ref[0] = contrib
        cnt_ref[...] = cnts

    @pl.when(mb > 0)
    def _():
        out_ref[0] += contrib
        cnt_ref[...] += cnts

    @pl.when(mb == nm - 1)
    def _():
        c = cnt_ref[...]
        out_ref[0] = out_ref[0] / (jnp.maximum(c, 1.0) + 1e-6)


def kernel(q, k, v, W_v):
    B, N, D = q.shape
    M = k.shape[1]
    MB = min(512, M)
    body = functools.partial(_body, n_total=N)
    return pl.pallas_call(
        body,
        grid=(B, M // MB),
        in_specs=[
            pl.BlockSpec((1, N, D), lambda b, m: (b, 0, 0)),
            pl.BlockSpec((1, MB, D), lambda b, m: (b, m, 0)),
            pl.BlockSpec((1, MB, D), lambda b, m: (b, m, 0)),
            pl.BlockSpec((D, D), lambda b, m: (0, 0)),
        ],
        out_specs=pl.BlockSpec((1, N, D), lambda b, m: (b, 0, 0)),
        out_shape=jax.ShapeDtypeStruct((B, N, D), jnp.float32),
        scratch_shapes=[pltpu.VMEM((N, 1), jnp.float32)],
    )(q, k, v, W_v)
